# baseline (speedup 1.0000x reference)
import functools

import jax
import jax.numpy as jnp
from jax import lax
from jax.experimental import pallas as pl
from jax.experimental.pallas import tpu as pltpu
from jax.experimental.pallas import tpu_sc as plsc

_N_EMBD = 2048
_NE = 16
_LB_SCALE = 0.01
_BLK = 1024
_SC_TOKENS = 1024
_NW = 32
_TPW = _SC_TOKENS // _NW
_SLAB = 16
_CH = _N_EMBD // 16


def _tc_gate_kernel(x_ref, w_ref, noise_ref, nw_ref,
                    weights_ref, ids_ref, psum_out_ref, psum_ref):
    i = pl.program_id(0)
    nblk = pl.num_programs(0)

    x = x_ref[...]
    w = w_ref[...]
    logits = jax.lax.dot_general(
        x, w, (((1,), (1,)), ((), ())), preferred_element_type=jnp.float32)

    noisy = logits + noise_ref[...] * nw_ref[...]

    cols = jax.lax.broadcasted_iota(jnp.int32, noisy.shape, 1)
    m1 = jnp.max(noisy, axis=1, keepdims=True)
    i1 = jnp.min(jnp.where(noisy == m1, cols, _NE), axis=1, keepdims=True)
    masked = jnp.where(cols == i1, -jnp.inf, noisy)
    m2 = jnp.max(masked, axis=1, keepdims=True)
    i2 = jnp.min(jnp.where(masked == m2, cols, _NE), axis=1, keepdims=True)

    e2 = jnp.exp(m2 - m1)
    denom = 1.0 + e2
    weights_ref[...] = (jnp.where(cols == i1, 1.0 / denom, 0.0)
                        + jnp.where(cols == i2, e2 / denom, 0.0))
    ids_ref[...] = jnp.concatenate([i1, i2], axis=1)

    mx = jnp.max(logits, axis=1, keepdims=True)
    ex = jnp.exp(logits - mx)
    p = ex / jnp.sum(ex, axis=1, keepdims=True)
    colsum = jnp.sum(p, axis=0, keepdims=True)

    @pl.when(i == 0)
    def _():
        psum_ref[...] = colsum

    @pl.when(i != 0)
    def _():
        psum_ref[...] = psum_ref[...] + colsum

    @pl.when(i == nblk - 1)
    def _():
        psum_out_ref[...] = psum_ref[...]


def _perm(v, idx):
    return v.at[idx].get(mode="promise_in_bounds")


def _bf16_round(v):
    u = lax.bitcast_convert_type(v, jnp.int32)
    r = u + jnp.int32(0x7FFF) + ((u >> 16) & 1)
    r = r & jnp.int32(-65536)
    return lax.bitcast_convert_type(r, jnp.float32)


def _all_reduce(v, op):
    iota = lax.iota(jnp.int32, 16)
    for s in (8, 4, 2, 1):
        v = op(v, _perm(v, iota ^ s))
    return v


def _sc_gate_kernel(x_hbm, w_hbm, noise_hbm, nw_hbm,
                    out_w, out_ids, out_psum,
                    xbuf, wbuf, noisebuf, nwbuf, wobuf, idobuf, psbuf,
                    *, s_base):
    wid = lax.axis_index("s") * 2 + lax.axis_index("c")
    pltpu.sync_copy(w_hbm, wbuf)
    pltpu.sync_copy(nw_hbm, nwbuf)
    nwv = nwbuf[...]
    iota = lax.iota(jnp.int32, 16)
    fzero = jnp.zeros((16,), jnp.float32)
    psbuf[...] = fzero

    def slab_body(slab, _):
        loc = wid * _TPW + slab * _SLAB
        tok = s_base + loc
        pltpu.sync_copy(x_hbm.at[pl.ds(tok, _SLAB)], xbuf)
        pltpu.sync_copy(noise_hbm.at[pl.ds(tok, _SLAB)], noisebuf)
        for tp in range(_SLAB // 2):
            t0, t1 = 2 * tp, 2 * tp + 1

            def body(c, carry, t0=t0, t1=t1):
                base = c * 16
                xa = _bf16_round(xbuf[t0, pl.ds(base, 16)])
                xb = _bf16_round(xbuf[t1, pl.ds(base, 16)])
                acca = list(carry[:_NE])
                accb = list(carry[_NE:])
                for e in range(_NE):
                    wv = wbuf[e, pl.ds(base, 16)]
                    acca[e] = acca[e] + xa * wv
                    accb[e] = accb[e] + xb * wv
                return tuple(acca) + tuple(accb)

            carry = lax.fori_loop(0, _CH, body, (fzero,) * (2 * _NE))

            for t, accs in ((t0, carry[:_NE]), (t1, carry[_NE:])):
                lv = fzero
                for e in range(_NE):
                    lv = jnp.where(iota == e, _all_reduce(accs[e], jnp.add),
                                   lv)
                noisy = lv + noisebuf[t, :] * nwv
                m1 = _all_reduce(noisy, jnp.maximum)
                i1 = _all_reduce(jnp.where(noisy == m1, iota, _NE),
                                 jnp.minimum)
                rest = iota != i1
                m2 = _all_reduce(jnp.where(rest, noisy, -jnp.inf),
                                 jnp.maximum)
                i2 = _all_reduce(jnp.where(rest & (noisy == m2), iota, _NE),
                                 jnp.minimum)
                sel = (iota == i1) | (iota == i2)
                ez = jnp.where(sel, jnp.exp(jnp.where(sel, noisy - m1, 0.0)),
                               0.0)
                wobuf[t, :] = ez / _all_reduce(ez, jnp.add)
                idobuf[t, :] = jnp.where(iota == 0, i1,
                                         jnp.where(iota == 1, i2, 0))
                mc = _all_reduce(lv, jnp.maximum)
                ec = jnp.exp(lv - mc)
                psbuf[...] = psbuf[...] + ec / _all_reduce(ec, jnp.add)
        pltpu.sync_copy(wobuf, out_w.at[pl.ds(loc, _SLAB)])
        pltpu.sync_copy(idobuf, out_ids.at[pl.ds(loc, _SLAB)])
        return ()

    lax.fori_loop(0, _TPW // _SLAB, slab_body, ())
    pltpu.sync_copy(psbuf, out_psum.at[wid])


def _loss_kernel(ptc_ref, psc_ref, loss_ref, *, token_count):
    tot = jnp.sum(ptc_ref[...], axis=0) + jnp.sum(psc_ref[...], axis=0)
    mean_p = tot / token_count
    dev = mean_p - (1.0 / _NE)
    loss_ref[...] = jnp.mean(dev * dev).reshape(1, 1) * _LB_SCALE


def kernel(x_flat, gate_W, noise_weight):
    token_count = x_flat.shape[0]
    tc_tokens = token_count - _SC_TOKENS
    noise = jax.random.normal(
        jax.random.key(12345), (token_count, _NE), dtype=jnp.float32)
    nw2d = noise_weight.reshape(1, _NE)

    sc_call = pl.kernel(
        functools.partial(_sc_gate_kernel, s_base=tc_tokens),
        out_type=[
            jax.ShapeDtypeStruct((_SC_TOKENS, _NE), jnp.float32),
            jax.ShapeDtypeStruct((_SC_TOKENS, _NE), jnp.int32),
            jax.ShapeDtypeStruct((_NW, _NE), jnp.float32),
        ],
        mesh=plsc.VectorSubcoreMesh(core_axis_name="c", subcore_axis_name="s"),
        scratch_types=[
            pltpu.VMEM((_SLAB, _N_EMBD), jnp.float32),
            pltpu.VMEM((_NE, _N_EMBD), jnp.float32),
            pltpu.VMEM((_SLAB, _NE), jnp.float32),
            pltpu.VMEM((_NE,), jnp.float32),
            pltpu.VMEM((_SLAB, _NE), jnp.float32),
            pltpu.VMEM((_SLAB, _NE), jnp.int32),
            pltpu.VMEM((_NE,), jnp.float32),
        ],
    )
    gate_W_r = _bf16_round(gate_W)
    w_sc, ids_sc, psc = sc_call(x_flat, gate_W_r, noise, noise_weight)

    grid = tc_tokens // _BLK
    w_tc, ids_tc, ptc = pl.pallas_call(
        _tc_gate_kernel,
        grid=(grid,),
        in_specs=[
            pl.BlockSpec((_BLK, _N_EMBD), lambda i: (i, 0)),
            pl.BlockSpec((_NE, _N_EMBD), lambda i: (0, 0)),
            pl.BlockSpec((_BLK, _NE), lambda i: (i, 0)),
            pl.BlockSpec((1, _NE), lambda i: (0, 0)),
        ],
        out_specs=[
            pl.BlockSpec((_BLK, _NE), lambda i: (i, 0)),
            pl.BlockSpec((_BLK, 2), lambda i: (i, 0)),
            pl.BlockSpec((1, _NE), lambda i: (0, 0)),
        ],
        out_shape=[
            jax.ShapeDtypeStruct((tc_tokens, _NE), jnp.float32),
            jax.ShapeDtypeStruct((tc_tokens, 2), jnp.int32),
            jax.ShapeDtypeStruct((1, _NE), jnp.float32),
        ],
        scratch_shapes=[pltpu.VMEM((1, _NE), jnp.float32)],
    )(x_flat, gate_W, noise, nw2d)

    loss = pl.pallas_call(
        functools.partial(_loss_kernel, token_count=token_count),
        out_shape=jax.ShapeDtypeStruct((1, 1), jnp.float32),
    )(ptc, psc)

    weights = jnp.concatenate([w_tc, w_sc], axis=0)
    ids = jnp.concatenate([ids_tc, ids_sc[:, :2]], axis=0)
    return weights, ids, loss[0, 0]


# speedup vs baseline: 1.6491x; 1.6491x over previous
import jax
import jax.numpy as jnp
from jax.experimental import pallas as pl
from jax.experimental.pallas import tpu as pltpu

_N_EMBD = 2048
_NUM_EXPERTS = 16
_LB_SCALE = 0.01
_NOISY_STD = 1.0
_BLK = 1024


def _gate_kernel(x_ref, w_ref, noise_ref, nw_ref,
                 weights_ref, ids_ref, loss_ref, psum_ref):
    i = pl.program_id(0)
    nblk = pl.num_programs(0)
    token_count = nblk * x_ref.shape[0]

    x = x_ref[...]
    w = w_ref[...]
    logits = jax.lax.dot_general(
        x, w, (((1,), (1,)), ((), ())), preferred_element_type=jnp.float32)

    noisy = logits + noise_ref[...] * nw_ref[...]

    cols = jax.lax.broadcasted_iota(jnp.int32, noisy.shape, 1)
    m1 = jnp.max(noisy, axis=1, keepdims=True)
    i1 = jnp.min(jnp.where(noisy == m1, cols, _NUM_EXPERTS),
                 axis=1, keepdims=True)
    masked = jnp.where(cols == i1, -jnp.inf, noisy)
    m2 = jnp.max(masked, axis=1, keepdims=True)
    i2 = jnp.min(jnp.where(masked == m2, cols, _NUM_EXPERTS),
                 axis=1, keepdims=True)

    e2 = jnp.exp(m2 - m1)
    denom = 1.0 + e2
    w1 = 1.0 / denom
    w2 = e2 / denom
    weights_ref[...] = (jnp.where(cols == i1, w1, 0.0)
                        + jnp.where(cols == i2, w2, 0.0))
    ids_ref[...] = jnp.concatenate([i1, i2], axis=1)

    mx = jnp.max(logits, axis=1, keepdims=True)
    ex = jnp.exp(logits - mx)
    p = ex / jnp.sum(ex, axis=1, keepdims=True)
    colsum = jnp.sum(p, axis=0, keepdims=True)

    @pl.when(i == 0)
    def _():
        psum_ref[...] = colsum

    @pl.when(i != 0)
    def _():
        psum_ref[...] = psum_ref[...] + colsum

    @pl.when(i == nblk - 1)
    def _():
        mean_p = psum_ref[...] / token_count
        dev = mean_p - (1.0 / _NUM_EXPERTS)
        loss_ref[...] = jnp.mean(dev * dev).reshape(1, 1) * _LB_SCALE


def kernel(x_flat, gate_W, noise_weight):
    token_count = x_flat.shape[0]
    num_experts = gate_W.shape[0]
    noise = jax.random.normal(
        jax.random.key(12345), (token_count, num_experts),
        dtype=jnp.float32) * _NOISY_STD
    nw = noise_weight.reshape(1, num_experts)

    grid = token_count // _BLK
    weights, ids, loss = pl.pallas_call(
        _gate_kernel,
        grid=(grid,),
        in_specs=[
            pl.BlockSpec((_BLK, _N_EMBD), lambda i: (i, 0)),
            pl.BlockSpec((num_experts, _N_EMBD), lambda i: (0, 0)),
            pl.BlockSpec((_BLK, num_experts), lambda i: (i, 0)),
            pl.BlockSpec((1, num_experts), lambda i: (0, 0)),
        ],
        out_specs=[
            pl.BlockSpec((_BLK, num_experts), lambda i: (i, 0)),
            pl.BlockSpec((_BLK, 2), lambda i: (i, 0)),
            pl.BlockSpec((1, 1), lambda i: (0, 0)),
        ],
        out_shape=[
            jax.ShapeDtypeStruct((token_count, num_experts), jnp.float32),
            jax.ShapeDtypeStruct((token_count, 2), jnp.int32),
            jax.ShapeDtypeStruct((1, 1), jnp.float32),
        ],
        scratch_shapes=[pltpu.VMEM((1, num_experts), jnp.float32)],
    )(x_flat, gate_W, noise, nw)
    return weights, ids, loss[0, 0]
